# Initial kernel scaffold; baseline (speedup 1.0000x reference)
import functools

import jax
import jax.numpy as jnp
from jax import lax
from jax.experimental import pallas as pl
from jax.experimental.pallas import tpu as pltpu
from jax.experimental.pallas import tpu_sc as plsc

N = 10000
E = 320000
N_PAD = 10240
NS = 16
EPC = E // NS
NPC = N_PAD // NS
SUB = 160
NSUB = NPC // SUB
EV = EPC // 16
L = 16
NEG = -3.0e38

_mesh = plsc.VectorSubcoreMesh(
    core_axis_name="c", subcore_axis_name="s", num_cores=1
)


@functools.partial(
    pl.kernel,
    out_type=jax.ShapeDtypeStruct((2 * N_PAD,), jnp.float32),
    mesh=_mesh,
    scratch_types=[
        pltpu.VMEM((N_PAD,), jnp.float32),
        pltpu.VMEM((N_PAD,), jnp.float32),
        pltpu.VMEM((N_PAD,), jnp.float32),
        pltpu.VMEM((N_PAD,), jnp.float32),
        pltpu.VMEM((N_PAD,), jnp.float32),
        pltpu.VMEM((N_PAD,), jnp.float32),
        pltpu.VMEM((N_PAD,), jnp.float32),
        pltpu.VMEM((N_PAD,), jnp.float32),
        pltpu.VMEM((EPC,), jnp.int32),
        pltpu.VMEM((EPC,), jnp.int32),
        pltpu.VMEM((NS, SUB), jnp.float32),
        pltpu.VMEM((SUB,), jnp.float32),
        pltpu.VMEM((SUB,), jnp.float32),
        pltpu.VMEM((SUB,), jnp.float32),
        pltpu.VMEM((2 * NPC,), jnp.float32),
        pltpu.VMEM((4, 128), jnp.float32),
        pltpu.VMEM((128,), jnp.float32),
        pltpu.VMEM((128,), jnp.float32),
        pltpu.VMEM((2, 128), jnp.float32),
        pltpu.VMEM_SHARED((NS, N_PAD), jnp.float32),
        pltpu.VMEM_SHARED((N_PAD,), jnp.float32),
        pltpu.VMEM_SHARED((NS, N_PAD), jnp.float32),
        pltpu.VMEM_SHARED((NS, N_PAD), jnp.float32),
        pltpu.VMEM_SHARED((NS, N_PAD), jnp.float32),
    ],
)
def _gat_sc(xT, srcE, dstE, W, att_s, att_d, WoT, out,
            pa, pd, m0, m1, mx, den, n0, n1, srcb, dstb,
            stage, rd, r0, r1, outc, wbuf, asb, adb, wob,
            SMX, SMG, SD, SN0, SN1):
    sid = lax.axis_index("s")

    pltpu.sync_copy(W, wbuf)
    pltpu.sync_copy(att_s, asb)
    pltpu.sync_copy(att_d, adb)
    pltpu.sync_copy(WoT, wob)

    def dot128(row_k, vec):
        acc = wbuf[row_k, pl.ds(0, L)] * vec[pl.ds(0, L)]
        for j in range(1, 8):
            acc = acc + wbuf[row_k, pl.ds(j * L, L)] * vec[pl.ds(j * L, L)]
        return jnp.sum(acc)

    def dotw(row_k, wrow):
        acc = wbuf[row_k, pl.ds(0, L)] * wob[wrow, pl.ds(0, L)]
        for j in range(1, 8):
            acc = acc + wbuf[row_k, pl.ds(j * L, L)] * wob[wrow, pl.ds(j * L, L)]
        return jnp.sum(acc)

    u = [dot128(k, asb) for k in range(4)]
    v = [dot128(k, adb) for k in range(4)]
    mc0 = [dotw(k, 0) for k in range(4)]
    mc1 = [dotw(k, 1) for k in range(4)]

    pltpu.sync_copy(xT.at[0], pa)
    pltpu.sync_copy(xT.at[1], pd)
    pltpu.sync_copy(xT.at[2], m0)
    pltpu.sync_copy(xT.at[3], m1)

    zero_v = jnp.zeros((L,), jnp.float32)
    neg_v = jnp.full((L,), NEG, jnp.float32)

    def proj_body(i, carry):
        s = pl.ds(i * L, L)
        x0 = pa[s]
        x1 = pd[s]
        x2 = m0[s]
        x3 = m1[s]
        pa[s] = x0 * u[0] + x1 * u[1] + x2 * u[2] + x3 * u[3]
        pd[s] = x0 * v[0] + x1 * v[1] + x2 * v[2] + x3 * v[3]
        m0[s] = x0 * mc0[0] + x1 * mc0[1] + x2 * mc0[2] + x3 * mc0[3]
        m1[s] = x0 * mc1[0] + x1 * mc1[1] + x2 * mc1[2] + x3 * mc1[3]
        mx[s] = neg_v
        den[s] = zero_v
        n0[s] = zero_v
        n1[s] = zero_v
        return carry

    lax.fori_loop(0, N_PAD // L, proj_body, 0)

    ebase = sid * EPC
    pltpu.sync_copy(srcE.at[pl.ds(ebase, EPC)], srcb)
    pltpu.sync_copy(dstE.at[pl.ds(ebase, EPC)], dstb)

    def p1_body(i, carry):
        s = pl.ds(i * L, L)
        si = srcb[s]
        di = dstb[s]
        z = plsc.load_gather(pa, [si]) + plsc.load_gather(pd, [di])
        e = jnp.where(z > 0, z, z * 0.2)
        cur = plsc.load_gather(mx, [di])
        plsc.store_scatter(mx, [di], jnp.maximum(cur, e))
        chk = plsc.load_gather(mx, [di])
        need0 = chk < e

        def retry_cond(need):
            return jnp.any(need)

        def retry_body(need):
            c2 = plsc.load_gather(mx, [di])
            plsc.store_scatter(mx, [di], jnp.maximum(c2, e), mask=need)
            c3 = plsc.load_gather(mx, [di])
            return jnp.logical_and(need, c3 < e)

        lax.while_loop(retry_cond, retry_body, need0)
        return carry

    lax.fori_loop(0, EV, p1_body, 0)

    pltpu.sync_copy(mx, SMX.at[sid])
    plsc.subcore_barrier()
    for sub in range(NSUB):
        off = sid * NPC + sub * SUB
        pltpu.sync_copy(SMX.at[:, pl.ds(off, SUB)], stage)

        def maxred_body(k, carry):
            sl = pl.ds(k * L, L)
            acc = stage[0, sl]
            for r in range(1, NS):
                acc = jnp.maximum(acc, stage[r, sl])
            rd[sl] = acc
            return carry

        lax.fori_loop(0, SUB // L, maxred_body, 0)
        pltpu.sync_copy(rd, SMG.at[pl.ds(off, SUB)])
    plsc.subcore_barrier()
    pltpu.sync_copy(SMG, mx)

    def p2_body(i, carry):
        s = pl.ds(i * L, L)
        si = srcb[s]
        di = dstb[s]
        z = plsc.load_gather(pa, [si]) + plsc.load_gather(pd, [di])
        e = jnp.where(z > 0, z, z * 0.2)
        mg = plsc.load_gather(mx, [di])
        ex = jnp.exp(e - mg)
        plsc.addupdate_scatter(den, [di], ex)
        w0 = plsc.load_gather(m0, [si])
        plsc.addupdate_scatter(n0, [di], ex * w0)
        w1 = plsc.load_gather(m1, [si])
        plsc.addupdate_scatter(n1, [di], ex * w1)
        return carry

    lax.fori_loop(0, EV, p2_body, 0)

    pltpu.sync_copy(den, SD.at[sid])
    pltpu.sync_copy(n0, SN0.at[sid])
    pltpu.sync_copy(n1, SN1.at[sid])
    plsc.subcore_barrier()

    for sub in range(NSUB):
        off = sid * NPC + sub * SUB
        for SRC_, buf in ((SD, rd), (SN0, r0), (SN1, r1)):
            pltpu.sync_copy(SRC_.at[:, pl.ds(off, SUB)], stage)

            def sumred_body(k, carry, buf=buf):
                sl = pl.ds(k * L, L)
                acc = stage[0, sl]
                for r in range(1, NS):
                    acc = acc + stage[r, sl]
                buf[sl] = acc
                return carry

            lax.fori_loop(0, SUB // L, sumred_body, 0)

        def fin_body(k, carry, sub=sub):
            sl = pl.ds(k * L, L)
            dsum = rd[sl] + 1e-16
            t0 = r0[sl] / dsum
            t1 = r1[sl] / dsum
            th0 = 1.0 - 2.0 / (jnp.exp(2.0 * t0) + 1.0)
            th1 = 1.0 - 2.0 / (jnp.exp(2.0 * t1) + 1.0)
            base = 2 * (sub * SUB + k * L)
            idx = 2 * lax.iota(jnp.int32, L) + base
            plsc.store_scatter(outc, [idx], th0)
            plsc.store_scatter(outc, [idx + 1], th1)
            return carry

        lax.fori_loop(0, SUB // L, fin_body, 0)

    pltpu.sync_copy(outc, out.at[pl.ds(sid * 2 * NPC, 2 * NPC)])


def kernel(x, edge_index, W, att_src, att_dst, W_out):
    xT = jnp.zeros((4, N_PAD), jnp.float32).at[:, :N].set(x.T)
    src = edge_index[0].astype(jnp.int32)
    dst = edge_index[1].astype(jnp.int32)
    out = _gat_sc(xT, src, dst, W, att_src, att_dst, W_out.T)
    return out[: 2 * N]


# speedup vs baseline: 55.6839x; 55.6839x over previous
import functools

import jax
import jax.numpy as jnp
from jax import lax
from jax.experimental import pallas as pl
from jax.experimental.pallas import tpu as pltpu
from jax.experimental.pallas import tpu_sc as plsc

N = 10000
E = 320000
N_PAD = 10240
NS = 16
EPC = E // NS
NPC = N_PAD // NS
SUB = 128
NSUB = NPC // SUB
EC = 2000
NCH = EPC // EC
L = 16
NEG = -3.0e38

_mesh = plsc.VectorSubcoreMesh(
    core_axis_name="c", subcore_axis_name="s", num_cores=1
)


@functools.partial(
    pl.kernel,
    out_type=jax.ShapeDtypeStruct((2 * N_PAD,), jnp.float32),
    mesh=_mesh,
    compiler_params=pltpu.CompilerParams(needs_layout_passes=False),
    scratch_types=[
        pltpu.VMEM((N_PAD,), jnp.float32),
        pltpu.VMEM((N_PAD,), jnp.float32),
        pltpu.VMEM((N_PAD,), jnp.float32),
        pltpu.VMEM((N_PAD,), jnp.float32),
        pltpu.VMEM((N_PAD,), jnp.float32),
        pltpu.VMEM((N_PAD,), jnp.float32),
        pltpu.VMEM((N_PAD,), jnp.float32),
        pltpu.VMEM((N_PAD,), jnp.float32),
        pltpu.VMEM((EC,), jnp.int32),
        pltpu.VMEM((EC,), jnp.int32),
        pltpu.VMEM((EC,), jnp.int32),
        pltpu.VMEM((EC,), jnp.int32),
        pltpu.VMEM((NS, SUB), jnp.float32),
        pltpu.VMEM((SUB,), jnp.float32),
        pltpu.VMEM((SUB,), jnp.float32),
        pltpu.VMEM((SUB,), jnp.float32),
        pltpu.VMEM((2 * NPC,), jnp.float32),
        pltpu.VMEM((4, 128), jnp.float32),
        pltpu.VMEM((128,), jnp.float32),
        pltpu.VMEM((128,), jnp.float32),
        pltpu.VMEM((2, 128), jnp.float32),
        pltpu.SemaphoreType.DMA,
        pltpu.SemaphoreType.DMA,
        pltpu.SemaphoreType.DMA,
        pltpu.SemaphoreType.DMA,
        pltpu.VMEM_SHARED((NS, N_PAD), jnp.float32),
        pltpu.VMEM_SHARED((N_PAD,), jnp.float32),
        pltpu.VMEM_SHARED((NS, N_PAD), jnp.float32),
        pltpu.VMEM_SHARED((NS, N_PAD), jnp.float32),
    ],
)
def _gat_sc(xT, srcE, dstE, W, att_s, att_d, WoT, out,
            pa, pd, m0, m1, mx, den, n0, n1,
            srcb0, srcb1, dstb0, dstb1,
            stage, rd, r0, r1, outc, wbuf, asb, adb, wob,
            sem_s0, sem_s1, sem_d0, sem_d1,
            SP, SMG, SN0, SN1):
    sid = lax.axis_index("s")

    pltpu.sync_copy(W, wbuf)
    pltpu.sync_copy(att_s, asb)
    pltpu.sync_copy(att_d, adb)
    pltpu.sync_copy(WoT, wob)

    def lane_sum(acc):
        tot = acc[0]
        for j in range(1, L):
            tot = tot + acc[j]
        return tot

    def dot128(row_k, vec):
        acc = wbuf[row_k, pl.ds(0, L)] * vec[pl.ds(0, L)]
        for j in range(1, 8):
            acc = acc + wbuf[row_k, pl.ds(j * L, L)] * vec[pl.ds(j * L, L)]
        return lane_sum(acc)

    def dotw(row_k, wrow):
        acc = wbuf[row_k, pl.ds(0, L)] * wob[wrow, pl.ds(0, L)]
        for j in range(1, 8):
            acc = acc + wbuf[row_k, pl.ds(j * L, L)] * wob[wrow, pl.ds(j * L, L)]
        return lane_sum(acc)

    u = [dot128(k, asb) for k in range(4)]
    v = [dot128(k, adb) for k in range(4)]
    mc0 = [dotw(k, 0) for k in range(4)]
    mc1 = [dotw(k, 1) for k in range(4)]

    pltpu.sync_copy(xT.at[0], pa)
    pltpu.sync_copy(xT.at[1], pd)
    pltpu.sync_copy(xT.at[2], m0)
    pltpu.sync_copy(xT.at[3], m1)

    zero_v = jnp.zeros((L,), jnp.float32)
    neg_v = jnp.full((L,), NEG, jnp.float32)

    def proj_body(i, carry):
        s = pl.ds(i * L, L)
        x0 = pa[s]
        x1 = pd[s]
        x2 = m0[s]
        x3 = m1[s]
        pa[s] = x0 * u[0] + x1 * u[1] + x2 * u[2] + x3 * u[3]
        pd[s] = x0 * v[0] + x1 * v[1] + x2 * v[2] + x3 * v[3]
        m0[s] = x0 * mc0[0] + x1 * mc0[1] + x2 * mc0[2] + x3 * mc0[3]
        m1[s] = x0 * mc1[0] + x1 * mc1[1] + x2 * mc1[2] + x3 * mc1[3]
        mx[s] = neg_v
        den[s] = zero_v
        n0[s] = zero_v
        n1[s] = zero_v
        return carry

    lax.fori_loop(0, N_PAD // L, proj_body, 0)

    sbufs = (srcb0, srcb1)
    dbufs = (dstb0, dstb1)
    ssems = (sem_s0, sem_s1)
    dsems = (sem_d0, sem_d1)

    def edge_pass(make_body):
        handles = [None, None]

        def start(c):
            slot = c % 2
            base = sid * EPC + c * EC
            hs = pltpu.async_copy(
                srcE.at[pl.ds(base, EC)], sbufs[slot], ssems[slot])
            hd = pltpu.async_copy(
                dstE.at[pl.ds(base, EC)], dbufs[slot], dsems[slot])
            handles[slot] = (hs, hd)

        start(0)
        for c in range(NCH):
            slot = c % 2
            if c + 1 < NCH:
                start(c + 1)
            hs, hd = handles[slot]
            hs.wait()
            hd.wait()
            lax.fori_loop(0, EC // L, make_body(sbufs[slot], dbufs[slot]), 0)

    def p1_make(srcb, dstb):
        def p1_body(i, carry):
            s = pl.ds(i * L, L)
            si = srcb[s]
            di = dstb[s]
            z = plsc.load_gather(pa, [si]) + plsc.load_gather(pd, [di])
            e = jnp.where(z > 0, z, z * 0.2)
            cur = plsc.load_gather(mx, [di])
            plsc.store_scatter(mx, [di], jnp.maximum(cur, e))
            chk = plsc.load_gather(mx, [di])
            need0 = chk < e

            def retry_cond(need):
                return jnp.any(need)

            def retry_body(need):
                c2 = plsc.load_gather(mx, [di])
                plsc.store_scatter(mx, [di], jnp.maximum(c2, e), mask=need)
                c3 = plsc.load_gather(mx, [di])
                return jnp.logical_and(need, c3 < e)

            lax.while_loop(retry_cond, retry_body, need0)
            return carry

        return p1_body

    edge_pass(p1_make)

    pltpu.sync_copy(mx, SP.at[sid])
    plsc.subcore_barrier()
    for sub in range(NSUB):
        off = sid * NPC + sub * SUB
        pltpu.sync_copy(SP.at[:, pl.ds(off, SUB)], stage)

        def maxred_body(k, carry):
            sl = pl.ds(k * L, L)
            acc = stage[0, sl]
            for r in range(1, NS):
                acc = jnp.maximum(acc, stage[r, sl])
            rd[sl] = acc
            return carry

        lax.fori_loop(0, SUB // L, maxred_body, 0)
        pltpu.sync_copy(rd, SMG.at[pl.ds(off, SUB)])
    plsc.subcore_barrier()
    pltpu.sync_copy(SMG, mx)

    def p2_make(srcb, dstb):
        def p2_body(i, carry):
            s = pl.ds(i * L, L)
            si = srcb[s]
            di = dstb[s]
            z = plsc.load_gather(pa, [si]) + plsc.load_gather(pd, [di])
            e = jnp.where(z > 0, z, z * 0.2)
            mg = plsc.load_gather(mx, [di])
            ex = jnp.exp(e - mg)
            plsc.addupdate_scatter(den, [di], ex)
            w0 = plsc.load_gather(m0, [si])
            plsc.addupdate_scatter(n0, [di], ex * w0)
            w1 = plsc.load_gather(m1, [si])
            plsc.addupdate_scatter(n1, [di], ex * w1)
            return carry

        return p2_body

    edge_pass(p2_make)

    pltpu.sync_copy(den, SP.at[sid])
    pltpu.sync_copy(n0, SN0.at[sid])
    pltpu.sync_copy(n1, SN1.at[sid])
    plsc.subcore_barrier()

    for sub in range(NSUB):
        off = sid * NPC + sub * SUB
        for SRC_, buf in ((SP, rd), (SN0, r0), (SN1, r1)):
            pltpu.sync_copy(SRC_.at[:, pl.ds(off, SUB)], stage)

            def sumred_body(k, carry, buf=buf):
                sl = pl.ds(k * L, L)
                acc = stage[0, sl]
                for r in range(1, NS):
                    acc = acc + stage[r, sl]
                buf[sl] = acc
                return carry

            lax.fori_loop(0, SUB // L, sumred_body, 0)

        def fin_body(k, carry, sub=sub):
            sl = pl.ds(k * L, L)
            dsum = rd[sl] + 1e-16
            t0 = r0[sl] / dsum
            t1 = r1[sl] / dsum
            th0 = 1.0 - 2.0 / (jnp.exp(2.0 * t0) + 1.0)
            th1 = 1.0 - 2.0 / (jnp.exp(2.0 * t1) + 1.0)
            base = 2 * (sub * SUB + k * L)
            idx = 2 * lax.iota(jnp.int32, L) + base
            plsc.store_scatter(outc, [idx], th0)
            plsc.store_scatter(outc, [idx + 1], th1)
            return carry

        lax.fori_loop(0, SUB // L, fin_body, 0)

    pltpu.sync_copy(outc, out.at[pl.ds(sid * 2 * NPC, 2 * NPC)])


def kernel(x, edge_index, W, att_src, att_dst, W_out):
    xT = jnp.zeros((4, N_PAD), jnp.float32).at[:, :N].set(x.T)
    src = edge_index[0].astype(jnp.int32)
    dst = edge_index[1].astype(jnp.int32)
    out = _gat_sc(xT, src, dst, W, att_src, att_dst, W_out.T)
    return out[: 2 * N]


# speedup vs baseline: 56.5130x; 1.0149x over previous
import functools

import jax
import jax.numpy as jnp
from jax import lax
from jax.experimental import pallas as pl
from jax.experimental.pallas import tpu as pltpu
from jax.experimental.pallas import tpu_sc as plsc

N = 10000
E = 320000
N_PAD = 10240
NS = 16
EPC = E // NS
NPC = N_PAD // NS
SUB = 128
NSUB = NPC // SUB
EC = 2000
NCH = EPC // EC
L = 16
NEG = -3.0e38

_mesh = plsc.VectorSubcoreMesh(
    core_axis_name="c", subcore_axis_name="s", num_cores=1
)


@functools.partial(
    pl.kernel,
    out_type=jax.ShapeDtypeStruct((2 * N_PAD,), jnp.float32),
    mesh=_mesh,
    compiler_params=pltpu.CompilerParams(needs_layout_passes=False),
    scratch_types=[
        pltpu.VMEM((N_PAD,), jnp.float32),
        pltpu.VMEM((N_PAD,), jnp.float32),
        pltpu.VMEM((N_PAD,), jnp.float32),
        pltpu.VMEM((N_PAD,), jnp.float32),
        pltpu.VMEM((N_PAD,), jnp.float32),
        pltpu.VMEM((N_PAD,), jnp.float32),
        pltpu.VMEM((N_PAD,), jnp.float32),
        pltpu.VMEM((N_PAD,), jnp.float32),
        pltpu.VMEM((EC,), jnp.int32),
        pltpu.VMEM((EC,), jnp.int32),
        pltpu.VMEM((EC,), jnp.int32),
        pltpu.VMEM((EC,), jnp.int32),
        pltpu.VMEM((NS, SUB), jnp.float32),
        pltpu.VMEM((SUB,), jnp.float32),
        pltpu.VMEM((SUB,), jnp.float32),
        pltpu.VMEM((SUB,), jnp.float32),
        pltpu.VMEM((2 * NPC,), jnp.float32),
        pltpu.VMEM((4, 128), jnp.float32),
        pltpu.VMEM((128,), jnp.float32),
        pltpu.VMEM((128,), jnp.float32),
        pltpu.VMEM((2, 128), jnp.float32),
        pltpu.SemaphoreType.DMA,
        pltpu.SemaphoreType.DMA,
        pltpu.SemaphoreType.DMA,
        pltpu.SemaphoreType.DMA,
        pltpu.VMEM_SHARED((NS, N_PAD), jnp.float32),
        pltpu.VMEM_SHARED((N_PAD,), jnp.float32),
        pltpu.VMEM_SHARED((NS, N_PAD), jnp.float32),
        pltpu.VMEM_SHARED((NS, N_PAD), jnp.float32),
    ],
)
def _gat_sc(xT, srcE, dstE, W, att_s, att_d, WoT, out,
            pa, pd, m0, m1, mx, den, n0, n1,
            srcb0, srcb1, dstb0, dstb1,
            stage, rd, r0, r1, outc, wbuf, asb, adb, wob,
            sem_s0, sem_s1, sem_d0, sem_d1,
            SP, SMG, SN0, SN1):
    sid = lax.axis_index("s")

    pltpu.sync_copy(W, wbuf)
    pltpu.sync_copy(att_s, asb)
    pltpu.sync_copy(att_d, adb)
    pltpu.sync_copy(WoT, wob)

    def lane_sum(acc):
        tot = acc[0]
        for j in range(1, L):
            tot = tot + acc[j]
        return tot

    def dot128(row_k, vec):
        acc = wbuf[row_k, pl.ds(0, L)] * vec[pl.ds(0, L)]
        for j in range(1, 8):
            acc = acc + wbuf[row_k, pl.ds(j * L, L)] * vec[pl.ds(j * L, L)]
        return lane_sum(acc)

    def dotw(row_k, wrow):
        acc = wbuf[row_k, pl.ds(0, L)] * wob[wrow, pl.ds(0, L)]
        for j in range(1, 8):
            acc = acc + wbuf[row_k, pl.ds(j * L, L)] * wob[wrow, pl.ds(j * L, L)]
        return lane_sum(acc)

    u = [dot128(k, asb) for k in range(4)]
    v = [dot128(k, adb) for k in range(4)]
    mc0 = [dotw(k, 0) for k in range(4)]
    mc1 = [dotw(k, 1) for k in range(4)]

    pltpu.sync_copy(xT.at[0], pa)
    pltpu.sync_copy(xT.at[1], pd)
    pltpu.sync_copy(xT.at[2], m0)
    pltpu.sync_copy(xT.at[3], m1)

    zero_v = jnp.zeros((L,), jnp.float32)
    neg_v = jnp.full((L,), NEG, jnp.float32)

    @plsc.parallel_loop(0, N_PAD // L, step=1, unroll=2)
    def _proj_body(i):
        s = pl.ds(i * L, L)
        x0 = pa[s]
        x1 = pd[s]
        x2 = m0[s]
        x3 = m1[s]
        pa[s] = x0 * u[0] + x1 * u[1] + x2 * u[2] + x3 * u[3]
        pd[s] = x0 * v[0] + x1 * v[1] + x2 * v[2] + x3 * v[3]
        m0[s] = x0 * mc0[0] + x1 * mc0[1] + x2 * mc0[2] + x3 * mc0[3]
        m1[s] = x0 * mc1[0] + x1 * mc1[1] + x2 * mc1[2] + x3 * mc1[3]
        mx[s] = neg_v
        den[s] = zero_v
        n0[s] = zero_v
        n1[s] = zero_v

    sbufs = (srcb0, srcb1)
    dbufs = (dstb0, dstb1)
    ssems = (sem_s0, sem_s1)
    dsems = (sem_d0, sem_d1)

    def edge_pass(make_body, parallel):
        handles = [None, None]

        def start(c):
            slot = c % 2
            base = sid * EPC + c * EC
            hs = pltpu.async_copy(
                srcE.at[pl.ds(base, EC)], sbufs[slot], ssems[slot])
            hd = pltpu.async_copy(
                dstE.at[pl.ds(base, EC)], dbufs[slot], dsems[slot])
            handles[slot] = (hs, hd)

        start(0)
        for c in range(NCH):
            slot = c % 2
            if c + 1 < NCH:
                start(c + 1)
            hs, hd = handles[slot]
            hs.wait()
            hd.wait()
            body = make_body(sbufs[slot], dbufs[slot])
            if parallel:
                def ploop_body(i, body=body):
                    body(i, 0)
                    return None
                plsc.parallel_loop(0, EC // L, step=1, unroll=2)(ploop_body)
            else:
                lax.fori_loop(0, EC // L, body, 0)

    def p1_make(srcb, dstb):
        def p1_body(i, carry):
            s = pl.ds(i * L, L)
            si = srcb[s]
            di = dstb[s]
            z = plsc.load_gather(pa, [si]) + plsc.load_gather(pd, [di])
            e = jnp.where(z > 0, z, z * 0.2)
            cur = plsc.load_gather(mx, [di])
            plsc.store_scatter(mx, [di], jnp.maximum(cur, e))
            chk = plsc.load_gather(mx, [di])
            need0 = chk < e

            def retry_cond(need):
                return jnp.any(need)

            def retry_body(need):
                c2 = plsc.load_gather(mx, [di])
                plsc.store_scatter(mx, [di], jnp.maximum(c2, e), mask=need)
                c3 = plsc.load_gather(mx, [di])
                return jnp.logical_and(need, c3 < e)

            lax.while_loop(retry_cond, retry_body, need0)
            return carry

        return p1_body

    edge_pass(p1_make, parallel=False)

    pltpu.sync_copy(mx, SP.at[sid])
    plsc.subcore_barrier()
    for sub in range(NSUB):
        off = sid * NPC + sub * SUB
        pltpu.sync_copy(SP.at[:, pl.ds(off, SUB)], stage)

        def maxred_body(k, carry):
            sl = pl.ds(k * L, L)
            acc = stage[0, sl]
            for r in range(1, NS):
                acc = jnp.maximum(acc, stage[r, sl])
            rd[sl] = acc
            return carry

        lax.fori_loop(0, SUB // L, maxred_body, 0)
        pltpu.sync_copy(rd, SMG.at[pl.ds(off, SUB)])
    plsc.subcore_barrier()
    pltpu.sync_copy(SMG, mx)

    def p2_make(srcb, dstb):
        def p2_body(i, carry):
            s = pl.ds(i * L, L)
            si = srcb[s]
            di = dstb[s]
            z = plsc.load_gather(pa, [si]) + plsc.load_gather(pd, [di])
            e = jnp.where(z > 0, z, z * 0.2)
            mg = plsc.load_gather(mx, [di])
            ex = jnp.exp(e - mg)
            plsc.addupdate_scatter(den, [di], ex)
            w0 = plsc.load_gather(m0, [si])
            plsc.addupdate_scatter(n0, [di], ex * w0)
            w1 = plsc.load_gather(m1, [si])
            plsc.addupdate_scatter(n1, [di], ex * w1)
            return carry

        return p2_body

    edge_pass(p2_make, parallel=False)

    pltpu.sync_copy(den, SP.at[sid])
    pltpu.sync_copy(n0, SN0.at[sid])
    pltpu.sync_copy(n1, SN1.at[sid])
    plsc.subcore_barrier()

    for sub in range(NSUB):
        off = sid * NPC + sub * SUB
        for SRC_, buf in ((SP, rd), (SN0, r0), (SN1, r1)):
            pltpu.sync_copy(SRC_.at[:, pl.ds(off, SUB)], stage)

            def sumred_body(k, carry, buf=buf):
                sl = pl.ds(k * L, L)
                acc = stage[0, sl]
                for r in range(1, NS):
                    acc = acc + stage[r, sl]
                buf[sl] = acc
                return carry

            lax.fori_loop(0, SUB // L, sumred_body, 0)

        def fin_body(k, carry, sub=sub):
            sl = pl.ds(k * L, L)
            dsum = rd[sl] + 1e-16
            t0 = r0[sl] / dsum
            t1 = r1[sl] / dsum
            th0 = 1.0 - 2.0 / (jnp.exp(2.0 * t0) + 1.0)
            th1 = 1.0 - 2.0 / (jnp.exp(2.0 * t1) + 1.0)
            base = 2 * (sub * SUB + k * L)
            idx = 2 * lax.iota(jnp.int32, L) + base
            plsc.store_scatter(outc, [idx], th0)
            plsc.store_scatter(outc, [idx + 1], th1)
            return carry

        lax.fori_loop(0, SUB // L, fin_body, 0)

    pltpu.sync_copy(outc, out.at[pl.ds(sid * 2 * NPC, 2 * NPC)])


def kernel(x, edge_index, W, att_src, att_dst, W_out):
    xT = jnp.zeros((4, N_PAD), jnp.float32).at[:, :N].set(x.T)
    src = edge_index[0].astype(jnp.int32)
    dst = edge_index[1].astype(jnp.int32)
    out = _gat_sc(xT, src, dst, W, att_src, att_dst, W_out.T)
    return out[: 2 * N]


# speedup vs baseline: 67.7355x; 1.1986x over previous
import functools

import jax
import jax.numpy as jnp
from jax import lax
from jax.experimental import pallas as pl
from jax.experimental.pallas import tpu as pltpu
from jax.experimental.pallas import tpu_sc as plsc

N = 10000
E = 320000
N_PAD = 10240
NS = 16
EPC = E // NS
NPC = N_PAD // NS
SUB = 128
NSUB = NPC // SUB
EC = 2000
NCH = EPC // EC
L = 16
NEG = -3.0e38
DEN_LO = 1e-30
FIN_HI = 3.4e38

_mesh = plsc.VectorSubcoreMesh(
    core_axis_name="c", subcore_axis_name="s", num_cores=1
)


def _soft_exp(x):
    y = x * 1.4426950408889634
    kf = y + jnp.where(y >= 0, 0.5, -0.5)
    k = kf.astype(jnp.int32)
    f = y - k.astype(jnp.float32)
    t = f * 0.6931471805599453
    p = t * (1.0 / 720.0) + (1.0 / 120.0)
    p = p * t + (1.0 / 24.0)
    p = p * t + (1.0 / 6.0)
    p = p * t + 0.5
    p = p * t + 1.0
    p = p * t + 1.0
    bits = plsc.bitcast(p, jnp.int32) + (k << 23)
    r = plsc.bitcast(bits, jnp.float32)
    r = jnp.where(x > 88.0, jnp.float32(jnp.inf), r)
    r = jnp.where(x < -87.0, jnp.float32(0.0), r)
    return r


@functools.partial(
    pl.kernel,
    out_type=jax.ShapeDtypeStruct((2 * N_PAD,), jnp.float32),
    mesh=_mesh,
    compiler_params=pltpu.CompilerParams(needs_layout_passes=False),
    scratch_types=[
        pltpu.VMEM((N_PAD,), jnp.float32),
        pltpu.VMEM((N_PAD,), jnp.float32),
        pltpu.VMEM((N_PAD,), jnp.float32),
        pltpu.VMEM((N_PAD,), jnp.float32),
        pltpu.VMEM((N_PAD,), jnp.float32),
        pltpu.VMEM((N_PAD,), jnp.float32),
        pltpu.VMEM((N_PAD,), jnp.float32),
        pltpu.VMEM((N_PAD,), jnp.float32),
        pltpu.VMEM((EC,), jnp.int32),
        pltpu.VMEM((EC,), jnp.int32),
        pltpu.VMEM((EC,), jnp.int32),
        pltpu.VMEM((EC,), jnp.int32),
        pltpu.VMEM((NS, SUB), jnp.float32),
        pltpu.VMEM((SUB,), jnp.float32),
        pltpu.VMEM((SUB,), jnp.float32),
        pltpu.VMEM((SUB,), jnp.float32),
        pltpu.VMEM((2 * NPC,), jnp.float32),
        pltpu.VMEM((4, 128), jnp.float32),
        pltpu.VMEM((128,), jnp.float32),
        pltpu.VMEM((128,), jnp.float32),
        pltpu.VMEM((2, 128), jnp.float32),
        pltpu.SemaphoreType.DMA,
        pltpu.SemaphoreType.DMA,
        pltpu.SemaphoreType.DMA,
        pltpu.SemaphoreType.DMA,
        pltpu.VMEM_SHARED((NS, N_PAD), jnp.float32),
        pltpu.VMEM_SHARED((N_PAD,), jnp.float32),
        pltpu.VMEM_SHARED((NS, N_PAD), jnp.float32),
        pltpu.VMEM_SHARED((NS, N_PAD), jnp.float32),
    ],
)
def _gat_sc(xT, srcE, dstE, W, att_s, att_d, WoT, out,
            pa, pd, m0, m1, mx, den, n0, n1,
            srcb0, srcb1, dstb0, dstb1,
            stage, rd, r0, r1, outc, wbuf, asb, adb, wob,
            sem_s0, sem_s1, sem_d0, sem_d1,
            SP, SMG, SN0, SN1):
    sid = lax.axis_index("s")

    pltpu.sync_copy(W, wbuf)
    pltpu.sync_copy(att_s, asb)
    pltpu.sync_copy(att_d, adb)
    pltpu.sync_copy(WoT, wob)

    def lane_sum(acc):
        tot = acc[0]
        for j in range(1, L):
            tot = tot + acc[j]
        return tot

    def dot128(row_k, vec):
        acc = wbuf[row_k, pl.ds(0, L)] * vec[pl.ds(0, L)]
        for j in range(1, 8):
            acc = acc + wbuf[row_k, pl.ds(j * L, L)] * vec[pl.ds(j * L, L)]
        return lane_sum(acc)

    def dotw(row_k, wrow):
        acc = wbuf[row_k, pl.ds(0, L)] * wob[wrow, pl.ds(0, L)]
        for j in range(1, 8):
            acc = acc + wbuf[row_k, pl.ds(j * L, L)] * wob[wrow, pl.ds(j * L, L)]
        return lane_sum(acc)

    u = [dot128(k, asb) for k in range(4)]
    v = [dot128(k, adb) for k in range(4)]
    mc0 = [dotw(k, 0) for k in range(4)]
    mc1 = [dotw(k, 1) for k in range(4)]

    pltpu.sync_copy(xT.at[0], pa)
    pltpu.sync_copy(xT.at[1], pd)
    pltpu.sync_copy(xT.at[2], m0)
    pltpu.sync_copy(xT.at[3], m1)

    zero_v = jnp.zeros((L,), jnp.float32)
    neg_v = jnp.full((L,), NEG, jnp.float32)

    @plsc.parallel_loop(0, N_PAD // L, step=1, unroll=2)
    def _proj_body(i):
        s = pl.ds(i * L, L)
        x0 = pa[s]
        x1 = pd[s]
        x2 = m0[s]
        x3 = m1[s]
        pa[s] = x0 * u[0] + x1 * u[1] + x2 * u[2] + x3 * u[3]
        pd[s] = x0 * v[0] + x1 * v[1] + x2 * v[2] + x3 * v[3]
        m0[s] = x0 * mc0[0] + x1 * mc0[1] + x2 * mc0[2] + x3 * mc0[3]
        m1[s] = x0 * mc1[0] + x1 * mc1[1] + x2 * mc1[2] + x3 * mc1[3]
        den[s] = zero_v
        n0[s] = zero_v
        n1[s] = zero_v

    sbufs = (srcb0, srcb1)
    dbufs = (dstb0, dstb1)
    ssems = (sem_s0, sem_s1)
    dsems = (sem_d0, sem_d1)

    def edge_pass(make_body):
        handles = [None, None]

        def start(c):
            slot = c % 2
            base = sid * EPC + c * EC
            hs = pltpu.async_copy(
                srcE.at[pl.ds(base, EC)], sbufs[slot], ssems[slot])
            hd = pltpu.async_copy(
                dstE.at[pl.ds(base, EC)], dbufs[slot], dsems[slot])
            handles[slot] = (hs, hd)

        start(0)
        for c in range(NCH):
            slot = c % 2
            if c + 1 < NCH:
                start(c + 1)
            hs, hd = handles[slot]
            hs.wait()
            hd.wait()
            lax.fori_loop(0, EC // L, make_body(sbufs[slot], dbufs[slot]), 0)

    def leaky(si, di):
        z = plsc.load_gather(pa, [si]) + plsc.load_gather(pd, [di])
        return jnp.where(z > 0, z, z * 0.2)

    def fast_make(srcb, dstb):
        def fast_body(i, carry):
            s = pl.ds(i * L, L)
            si = srcb[s]
            di = dstb[s]
            ex = _soft_exp(leaky(si, di))
            plsc.addupdate_scatter(den, [di], ex)
            w0 = plsc.load_gather(m0, [si])
            plsc.addupdate_scatter(n0, [di], ex * w0)
            w1 = plsc.load_gather(m1, [si])
            plsc.addupdate_scatter(n1, [di], ex * w1)
            return carry

        return fast_body

    edge_pass(fast_make)

    pltpu.sync_copy(den, SP.at[sid])
    pltpu.sync_copy(n0, SN0.at[sid])
    pltpu.sync_copy(n1, SN1.at[sid])
    plsc.subcore_barrier()

    def reduce_sums(sub):
        off = sid * NPC + sub * SUB
        for SRC_, buf in ((SP, rd), (SN0, r0), (SN1, r1)):
            pltpu.sync_copy(SRC_.at[:, pl.ds(off, SUB)], stage)

            def sumred_body(k, carry, buf=buf):
                sl = pl.ds(k * L, L)
                acc = stage[0, sl]
                for r in range(1, NS):
                    acc = acc + stage[r, sl]
                buf[sl] = acc
                return carry

            lax.fori_loop(0, SUB // L, sumred_body, 0)
        return off

    badcnt = zero_v
    for sub in range(NSUB):
        off = reduce_sums(sub)

        def fin_fast(k, bad, sub=sub, off=off):
            sl = pl.ds(k * L, L)
            dsum = rd[sl]
            a0 = r0[sl]
            a1 = r1[sl]
            ok = (dsum >= DEN_LO) & (dsum <= FIN_HI)
            ok = ok & (jnp.abs(a0) <= FIN_HI) & (jnp.abs(a1) <= FIN_HI)
            nodeid = off + k * L + lax.iota(jnp.int32, L)
            bad_v = jnp.logical_and(nodeid < N, jnp.logical_not(ok))
            bad = bad + jnp.where(bad_v, 1.0, 0.0)
            dsafe = jnp.where(dsum >= DEN_LO, dsum, 1.0)
            t0 = a0 / dsafe
            t1 = a1 / dsafe
            th0 = 1.0 - 2.0 / (_soft_exp(2.0 * t0) + 1.0)
            th1 = 1.0 - 2.0 / (_soft_exp(2.0 * t1) + 1.0)
            base = 2 * (sub * SUB + k * L)
            idx = 2 * lax.iota(jnp.int32, L) + base
            plsc.store_scatter(outc, [idx], th0)
            plsc.store_scatter(outc, [idx + 1], th1)
            return bad

        badcnt = lax.fori_loop(0, SUB // L, fin_fast, badcnt)

    rd[pl.ds(0, L)] = badcnt
    pltpu.sync_copy(rd.at[pl.ds(0, L)], SMG.at[pl.ds(sid * L, L)])
    plsc.subcore_barrier()
    pltpu.sync_copy(SMG.at[pl.ds(0, NS * L)], mx.at[pl.ds(0, NS * L)])
    tot = mx[pl.ds(0, L)]
    for r in range(1, NS):
        tot = tot + mx[pl.ds(r * L, L)]
    any_bad = lane_sum(tot) > 0.0

    def p1_body(i, carry):
        s = pl.ds(i * L, L)
        si = srcb0[s]
        di = dstb0[s]
        e = leaky(si, di)
        cur = plsc.load_gather(mx, [di])
        plsc.store_scatter(mx, [di], jnp.maximum(cur, e))
        chk = plsc.load_gather(mx, [di])
        need0 = chk < e

        def retry_cond(need):
            return jnp.any(need)

        def retry_body(need):
            c2 = plsc.load_gather(mx, [di])
            plsc.store_scatter(mx, [di], jnp.maximum(c2, e), mask=need)
            c3 = plsc.load_gather(mx, [di])
            return jnp.logical_and(need, c3 < e)

        lax.while_loop(retry_cond, retry_body, need0)
        return carry

    def p2_body(i, carry):
        s = pl.ds(i * L, L)
        si = srcb0[s]
        di = dstb0[s]
        e = leaky(si, di)
        mg = plsc.load_gather(mx, [di])
        ex = _soft_exp(e - mg)
        plsc.addupdate_scatter(den, [di], ex)
        w0 = plsc.load_gather(m0, [si])
        plsc.addupdate_scatter(n0, [di], ex * w0)
        w1 = plsc.load_gather(m1, [si])
        plsc.addupdate_scatter(n1, [di], ex * w1)
        return carry

    def chunked(body):
        def chunk_body(c, carry):
            base = sid * EPC + c * EC
            pltpu.sync_copy(srcE.at[pl.ds(base, EC)], srcb0)
            pltpu.sync_copy(dstE.at[pl.ds(base, EC)], dstb0)
            lax.fori_loop(0, EC // L, body, 0)
            return carry

        lax.fori_loop(0, NCH, chunk_body, 0)

    def fallback():
        def init_body(i, carry):
            s = pl.ds(i * L, L)
            mx[s] = neg_v
            den[s] = zero_v
            n0[s] = zero_v
            n1[s] = zero_v
            return carry

        lax.fori_loop(0, N_PAD // L, init_body, 0)

        chunked(p1_body)

        pltpu.sync_copy(mx, SP.at[sid])
        plsc.subcore_barrier()
        for sub in range(NSUB):
            off = sid * NPC + sub * SUB
            pltpu.sync_copy(SP.at[:, pl.ds(off, SUB)], stage)

            def maxred_body(k, carry):
                sl = pl.ds(k * L, L)
                acc = stage[0, sl]
                for r in range(1, NS):
                    acc = jnp.maximum(acc, stage[r, sl])
                rd[sl] = acc
                return carry

            lax.fori_loop(0, SUB // L, maxred_body, 0)
            pltpu.sync_copy(rd, SMG.at[pl.ds(off, SUB)])
        plsc.subcore_barrier()
        pltpu.sync_copy(SMG, mx)

        chunked(p2_body)

        pltpu.sync_copy(den, SP.at[sid])
        pltpu.sync_copy(n0, SN0.at[sid])
        pltpu.sync_copy(n1, SN1.at[sid])
        plsc.subcore_barrier()

        for sub in range(NSUB):
            reduce_sums(sub)

            def fin_exact(k, carry, sub=sub):
                sl = pl.ds(k * L, L)
                dsum = rd[sl] + 1e-16
                t0 = r0[sl] / dsum
                t1 = r1[sl] / dsum
                th0 = 1.0 - 2.0 / (_soft_exp(2.0 * t0) + 1.0)
                th1 = 1.0 - 2.0 / (_soft_exp(2.0 * t1) + 1.0)
                base = 2 * (sub * SUB + k * L)
                idx = 2 * lax.iota(jnp.int32, L) + base
                plsc.store_scatter(outc, [idx], th0)
                plsc.store_scatter(outc, [idx + 1], th1)
                return carry

            lax.fori_loop(0, SUB // L, fin_exact, 0)

    lax.cond(any_bad, fallback, lambda: None)

    pltpu.sync_copy(outc, out.at[pl.ds(sid * 2 * NPC, 2 * NPC)])


def kernel(x, edge_index, W, att_src, att_dst, W_out):
    xT = jnp.zeros((4, N_PAD), jnp.float32).at[:, :N].set(x.T)
    src = edge_index[0].astype(jnp.int32)
    dst = edge_index[1].astype(jnp.int32)
    out = _gat_sc(xT, src, dst, W, att_src, att_dst, W_out.T)
    return out[: 2 * N]


# speedup vs baseline: 107.2699x; 1.5837x over previous
import functools

import jax
import jax.numpy as jnp
from jax import lax
from jax.experimental import pallas as pl
from jax.experimental.pallas import tpu as pltpu
from jax.experimental.pallas import tpu_sc as plsc

N = 10000
E = 320000
N_PAD = 10240
NS = 16
EPC = E // NS
NPC = N_PAD // NS
SUB = 128
NSUB = NPC // SUB
EC = 2000
NCH = EPC // EC
UNR = 5
L = 16
NEG = -3.0e38
DEN_LO = 1e-30
FIN_HI = 3.4e38

_mesh = plsc.VectorSubcoreMesh(
    core_axis_name="c", subcore_axis_name="s", num_cores=1
)


def _soft_exp(x):
    y = x * 1.4426950408889634
    kf = y + jnp.where(y >= 0, 0.5, -0.5)
    k = kf.astype(jnp.int32)
    f = y - k.astype(jnp.float32)
    t = f * 0.6931471805599453
    p = t * (1.0 / 720.0) + (1.0 / 120.0)
    p = p * t + (1.0 / 24.0)
    p = p * t + (1.0 / 6.0)
    p = p * t + 0.5
    p = p * t + 1.0
    p = p * t + 1.0
    bits = plsc.bitcast(p, jnp.int32) + (k << 23)
    r = plsc.bitcast(bits, jnp.float32)
    r = jnp.where(x > 88.0, jnp.float32(jnp.inf), r)
    r = jnp.where(x < -87.0, jnp.float32(0.0), r)
    return r


@functools.partial(
    pl.kernel,
    out_type=jax.ShapeDtypeStruct((2 * N_PAD,), jnp.float32),
    mesh=_mesh,
    compiler_params=pltpu.CompilerParams(needs_layout_passes=False),
    scratch_types=[
        pltpu.VMEM((N_PAD,), jnp.float32),
        pltpu.VMEM((N_PAD,), jnp.float32),
        pltpu.VMEM((N_PAD,), jnp.float32),
        pltpu.VMEM((N_PAD,), jnp.float32),
        pltpu.VMEM((N_PAD,), jnp.float32),
        pltpu.VMEM((N_PAD,), jnp.float32),
        pltpu.VMEM((N_PAD,), jnp.float32),
        pltpu.VMEM((N_PAD,), jnp.float32),
        pltpu.VMEM((EC,), jnp.int32),
        pltpu.VMEM((EC,), jnp.int32),
        pltpu.VMEM((EC,), jnp.int32),
        pltpu.VMEM((EC,), jnp.int32),
        pltpu.VMEM((NS, SUB), jnp.float32),
        pltpu.VMEM((SUB,), jnp.float32),
        pltpu.VMEM((SUB,), jnp.float32),
        pltpu.VMEM((SUB,), jnp.float32),
        pltpu.VMEM((2 * NPC,), jnp.float32),
        pltpu.VMEM((4, 128), jnp.float32),
        pltpu.VMEM((128,), jnp.float32),
        pltpu.VMEM((128,), jnp.float32),
        pltpu.VMEM((2, 128), jnp.float32),
        pltpu.SemaphoreType.DMA,
        pltpu.SemaphoreType.DMA,
        pltpu.SemaphoreType.DMA,
        pltpu.SemaphoreType.DMA,
        pltpu.VMEM_SHARED((NS, N_PAD), jnp.float32),
        pltpu.VMEM_SHARED((N_PAD,), jnp.float32),
        pltpu.VMEM_SHARED((NS, N_PAD), jnp.float32),
        pltpu.VMEM_SHARED((NS, N_PAD), jnp.float32),
    ],
)
def _gat_sc(xT, srcE, dstE, W, att_s, att_d, WoT, out,
            pa, pd, m0, m1, mx, den, n0, n1,
            srcb0, srcb1, dstb0, dstb1,
            stage, rd, r0, r1, outc, wbuf, asb, adb, wob,
            sem_s0, sem_s1, sem_d0, sem_d1,
            SP, SMG, SN0, SN1):
    sid = lax.axis_index("s")

    pltpu.sync_copy(W, wbuf)
    pltpu.sync_copy(att_s, asb)
    pltpu.sync_copy(att_d, adb)
    pltpu.sync_copy(WoT, wob)

    def lane_sum(acc):
        tot = acc[0]
        for j in range(1, L):
            tot = tot + acc[j]
        return tot

    def dot128(row_k, vec):
        acc = wbuf[row_k, pl.ds(0, L)] * vec[pl.ds(0, L)]
        for j in range(1, 8):
            acc = acc + wbuf[row_k, pl.ds(j * L, L)] * vec[pl.ds(j * L, L)]
        return lane_sum(acc)

    def dotw(row_k, wrow):
        acc = wbuf[row_k, pl.ds(0, L)] * wob[wrow, pl.ds(0, L)]
        for j in range(1, 8):
            acc = acc + wbuf[row_k, pl.ds(j * L, L)] * wob[wrow, pl.ds(j * L, L)]
        return lane_sum(acc)

    u = [dot128(k, asb) for k in range(4)]
    v = [dot128(k, adb) for k in range(4)]
    mc0 = [dotw(k, 0) for k in range(4)]
    mc1 = [dotw(k, 1) for k in range(4)]

    pltpu.sync_copy(xT.at[0], pa)
    pltpu.sync_copy(xT.at[1], pd)
    pltpu.sync_copy(xT.at[2], m0)
    pltpu.sync_copy(xT.at[3], m1)

    zero_v = jnp.zeros((L,), jnp.float32)
    neg_v = jnp.full((L,), NEG, jnp.float32)

    @plsc.parallel_loop(0, N_PAD // L, step=1, unroll=2)
    def _proj_body(i):
        s = pl.ds(i * L, L)
        x0 = pa[s]
        x1 = pd[s]
        x2 = m0[s]
        x3 = m1[s]
        pa[s] = x0 * u[0] + x1 * u[1] + x2 * u[2] + x3 * u[3]
        pd[s] = x0 * v[0] + x1 * v[1] + x2 * v[2] + x3 * v[3]
        m0[s] = x0 * mc0[0] + x1 * mc0[1] + x2 * mc0[2] + x3 * mc0[3]
        m1[s] = x0 * mc1[0] + x1 * mc1[1] + x2 * mc1[2] + x3 * mc1[3]
        den[s] = zero_v
        n0[s] = zero_v
        n1[s] = zero_v

    sbufs = (srcb0, srcb1)
    dbufs = (dstb0, dstb1)
    ssems = (sem_s0, sem_s1)
    dsems = (sem_d0, sem_d1)

    def leaky(si, di):
        z = plsc.load_gather(pa, [si]) + plsc.load_gather(pd, [di])
        return jnp.where(z > 0, z, z * 0.2)

    def fast_body_for(slot):
        srcb = sbufs[slot]
        dstb = dbufs[slot]

        def fast_body(i, carry):
            dis, zs, w0s, w1s = [], [], [], []
            for uu in range(UNR):
                s = pl.ds(i * (L * UNR) + uu * L, L)
                si = srcb[s]
                di = dstb[s]
                z = plsc.load_gather(pa, [si]) + plsc.load_gather(pd, [di])
                w0s.append(plsc.load_gather(m0, [si]))
                w1s.append(plsc.load_gather(m1, [si]))
                dis.append(di)
                zs.append(z)
            exs = [_soft_exp(jnp.where(z > 0, z, z * 0.2)) for z in zs]
            for uu in range(UNR):
                plsc.addupdate_scatter(den, [dis[uu]], exs[uu])
                plsc.addupdate_scatter(n0, [dis[uu]], exs[uu] * w0s[uu])
                plsc.addupdate_scatter(n1, [dis[uu]], exs[uu] * w1s[uu])
            return carry

        return fast_body

    def issue(ch, slot):
        base = sid * EPC + ch * EC
        pltpu.async_copy(srcE.at[pl.ds(base, EC)], sbufs[slot], ssems[slot])
        pltpu.async_copy(dstE.at[pl.ds(base, EC)], dbufs[slot], dsems[slot])

    def drain(slot):
        pltpu.make_async_copy(
            srcE.at[pl.ds(0, EC)], sbufs[slot], ssems[slot]).wait()
        pltpu.make_async_copy(
            dstE.at[pl.ds(0, EC)], dbufs[slot], dsems[slot]).wait()

    issue(0, 0)
    issue(1, 1)

    def pair_body(cc, carry):
        for slot in (0, 1):
            drain(slot)
            lax.fori_loop(0, EC // (L * UNR), fast_body_for(slot), 0)
            issue(2 * cc + slot + 2, slot)
        return carry

    lax.fori_loop(0, NCH // 2, pair_body, 0)
    drain(0)
    drain(1)

    pltpu.sync_copy(den, SP.at[sid])
    pltpu.sync_copy(n0, SN0.at[sid])
    pltpu.sync_copy(n1, SN1.at[sid])
    plsc.subcore_barrier()

    def reduce_sums(sub):
        off = sid * NPC + sub * SUB
        for SRC_, buf in ((SP, rd), (SN0, r0), (SN1, r1)):
            pltpu.sync_copy(SRC_.at[:, pl.ds(off, SUB)], stage)

            def sumred_body(k, carry, buf=buf):
                sl = pl.ds(k * L, L)
                acc = stage[0, sl]
                for r in range(1, NS):
                    acc = acc + stage[r, sl]
                buf[sl] = acc
                return carry

            lax.fori_loop(0, SUB // L, sumred_body, 0)
        return off

    badcnt = zero_v
    for sub in range(NSUB):
        off = reduce_sums(sub)

        def fin_fast(k, bad, sub=sub, off=off):
            sl = pl.ds(k * L, L)
            dsum = rd[sl]
            a0 = r0[sl]
            a1 = r1[sl]
            ok = (dsum >= DEN_LO) & (dsum <= FIN_HI)
            ok = ok & (jnp.abs(a0) <= FIN_HI) & (jnp.abs(a1) <= FIN_HI)
            nodeid = off + k * L + lax.iota(jnp.int32, L)
            bad_v = jnp.logical_and(nodeid < N, jnp.logical_not(ok))
            bad = bad + jnp.where(bad_v, 1.0, 0.0)
            dsafe = jnp.where(dsum >= DEN_LO, dsum, 1.0)
            t0 = a0 / dsafe
            t1 = a1 / dsafe
            th0 = 1.0 - 2.0 / (_soft_exp(2.0 * t0) + 1.0)
            th1 = 1.0 - 2.0 / (_soft_exp(2.0 * t1) + 1.0)
            base = 2 * (sub * SUB + k * L)
            idx = 2 * lax.iota(jnp.int32, L) + base
            plsc.store_scatter(outc, [idx], th0)
            plsc.store_scatter(outc, [idx + 1], th1)
            return bad

        badcnt = lax.fori_loop(0, SUB // L, fin_fast, badcnt)

    rd[pl.ds(0, L)] = badcnt
    pltpu.sync_copy(rd.at[pl.ds(0, L)], SMG.at[pl.ds(sid * L, L)])
    plsc.subcore_barrier()
    pltpu.sync_copy(SMG.at[pl.ds(0, NS * L)], mx.at[pl.ds(0, NS * L)])
    tot = mx[pl.ds(0, L)]
    for r in range(1, NS):
        tot = tot + mx[pl.ds(r * L, L)]
    any_bad = lane_sum(tot) > 0.0

    def p1_body(i, carry):
        s = pl.ds(i * L, L)
        si = srcb0[s]
        di = dstb0[s]
        e = leaky(si, di)
        cur = plsc.load_gather(mx, [di])
        plsc.store_scatter(mx, [di], jnp.maximum(cur, e))
        chk = plsc.load_gather(mx, [di])
        need0 = chk < e

        def retry_cond(need):
            return jnp.any(need)

        def retry_body(need):
            c2 = plsc.load_gather(mx, [di])
            plsc.store_scatter(mx, [di], jnp.maximum(c2, e), mask=need)
            c3 = plsc.load_gather(mx, [di])
            return jnp.logical_and(need, c3 < e)

        lax.while_loop(retry_cond, retry_body, need0)
        return carry

    def p2_body(i, carry):
        s = pl.ds(i * L, L)
        si = srcb0[s]
        di = dstb0[s]
        e = leaky(si, di)
        mg = plsc.load_gather(mx, [di])
        ex = _soft_exp(e - mg)
        plsc.addupdate_scatter(den, [di], ex)
        w0 = plsc.load_gather(m0, [si])
        plsc.addupdate_scatter(n0, [di], ex * w0)
        w1 = plsc.load_gather(m1, [si])
        plsc.addupdate_scatter(n1, [di], ex * w1)
        return carry

    def chunked(body):
        def chunk_body(c, carry):
            base = sid * EPC + c * EC
            pltpu.sync_copy(srcE.at[pl.ds(base, EC)], srcb0)
            pltpu.sync_copy(dstE.at[pl.ds(base, EC)], dstb0)
            lax.fori_loop(0, EC // L, body, 0)
            return carry

        lax.fori_loop(0, NCH, chunk_body, 0)

    def fallback():
        def init_body(i, carry):
            s = pl.ds(i * L, L)
            mx[s] = neg_v
            den[s] = zero_v
            n0[s] = zero_v
            n1[s] = zero_v
            return carry

        lax.fori_loop(0, N_PAD // L, init_body, 0)

        chunked(p1_body)

        pltpu.sync_copy(mx, SP.at[sid])
        plsc.subcore_barrier()
        for sub in range(NSUB):
            off = sid * NPC + sub * SUB
            pltpu.sync_copy(SP.at[:, pl.ds(off, SUB)], stage)

            def maxred_body(k, carry):
                sl = pl.ds(k * L, L)
                acc = stage[0, sl]
                for r in range(1, NS):
                    acc = jnp.maximum(acc, stage[r, sl])
                rd[sl] = acc
                return carry

            lax.fori_loop(0, SUB // L, maxred_body, 0)
            pltpu.sync_copy(rd, SMG.at[pl.ds(off, SUB)])
        plsc.subcore_barrier()
        pltpu.sync_copy(SMG, mx)

        chunked(p2_body)

        pltpu.sync_copy(den, SP.at[sid])
        pltpu.sync_copy(n0, SN0.at[sid])
        pltpu.sync_copy(n1, SN1.at[sid])
        plsc.subcore_barrier()

        for sub in range(NSUB):
            reduce_sums(sub)

            def fin_exact(k, carry, sub=sub):
                sl = pl.ds(k * L, L)
                dsum = rd[sl] + 1e-16
                t0 = r0[sl] / dsum
                t1 = r1[sl] / dsum
                th0 = 1.0 - 2.0 / (_soft_exp(2.0 * t0) + 1.0)
                th1 = 1.0 - 2.0 / (_soft_exp(2.0 * t1) + 1.0)
                base = 2 * (sub * SUB + k * L)
                idx = 2 * lax.iota(jnp.int32, L) + base
                plsc.store_scatter(outc, [idx], th0)
                plsc.store_scatter(outc, [idx + 1], th1)
                return carry

            lax.fori_loop(0, SUB // L, fin_exact, 0)

    lax.cond(any_bad, fallback, lambda: None)

    pltpu.sync_copy(outc, out.at[pl.ds(sid * 2 * NPC, 2 * NPC)])


def kernel(x, edge_index, W, att_src, att_dst, W_out):
    xT = jnp.zeros((4, N_PAD), jnp.float32).at[:, :N].set(x.T)
    ei = jnp.zeros((2, E + 2 * EC), jnp.int32).at[:, :E].set(
        edge_index.astype(jnp.int32))
    out = _gat_sc(xT, ei[0], ei[1], W, att_src, att_dst, W_out.T)
    return out[: 2 * N]
